# Initial kernel scaffold; baseline (speedup 1.0000x reference)
import jax
import jax.numpy as jnp
from jax.experimental import pallas as pl


def kernel(x, Wg, W_up, b_up, W_down, b_down):
    raise NotImplementedError("write your pallas kernel here")


# speedup vs baseline: 1.7124x; 1.7124x over previous
import functools

import jax
import jax.numpy as jnp
from jax.experimental import pallas as pl

S = 2048
D_MODEL = 768
D_FF = 3072
E = 16
EPAD = 128
FC = 768
NF = D_FF // FC
SC = 256
NS = S // SC


def _router_body(x_ref, wg_ref, wd_ref):
    x = x_ref[...]
    wg = wg_ref[...]
    logits = jax.lax.dot_general(
        x, wg, (((1,), (1,)), ((), ())), preferred_element_type=jnp.float32)
    col = jax.lax.broadcasted_iota(jnp.int32, (S, EPAD), 1)
    neg = jnp.float32(-1e30)
    lm = jnp.where(col < E, logits, neg)
    m0 = jnp.max(lm, axis=1, keepdims=True)
    i0 = jnp.min(jnp.where(lm == m0, col, jnp.int32(EPAD)), axis=1,
                 keepdims=True)
    l2 = jnp.where(col == i0, neg, lm)
    m1 = jnp.max(l2, axis=1, keepdims=True)
    i1 = jnp.min(jnp.where(l2 == m1, col, jnp.int32(EPAD)), axis=1,
                 keepdims=True)
    z = jnp.exp(m1 - m0)
    denom = 1.0 + z
    w0 = 1.0 / denom
    w1 = z / denom
    wd_ref[...] = jnp.where(col == i0, w0, jnp.where(col == i1, w1, 0.0))


def _ffn_body(wd_ref, x_ref, wu_ref, bu_ref, wdn_ref, bdn_ref, out_ref):
    e = pl.program_id(0)
    f = pl.program_id(1)
    s = pl.program_id(2)
    sblk = pl.ds(s * SC, SC)

    xs = x_ref[sblk, :].astype(jnp.bfloat16)
    wu = wu_ref[0].astype(jnp.bfloat16)
    h_pre = jax.lax.dot_general(
        xs, wu, (((1,), (1,)), ((), ())), preferred_element_type=jnp.float32)
    h_pre = h_pre + bu_ref[0, 0, :][None, :]
    h = 0.5 * h_pre * (1.0 + jax.lax.erf(h_pre * 0.7071067811865476))
    wdn = wdn_ref[0].astype(jnp.bfloat16)
    y = jax.lax.dot_general(
        h.astype(jnp.bfloat16), wdn, (((1,), (1,)), ((), ())),
        preferred_element_type=jnp.float32)

    col = jax.lax.broadcasted_iota(jnp.int32, (SC, EPAD), 1)
    wd = wd_ref[...]
    scale = jnp.sum(jnp.where(col == e, wd, 0.0), axis=1)[:, None]

    @pl.when((e == 0) & (f == 0))
    def _():
        out_ref[sblk, :] = jnp.zeros((SC, D_MODEL), jnp.float32)

    out_ref[sblk, :] += scale * y

    @pl.when(f == 0)
    def _():
        out_ref[sblk, :] += scale * bdn_ref[0, 0, :][None, :]


@functools.partial(jax.jit, static_argnames=())
def _run(x2, wg_pad, w_up, bu3, w_down, bdn3):
    wd = pl.pallas_call(
        _router_body,
        out_shape=jax.ShapeDtypeStruct((S, EPAD), jnp.float32),
    )(x2, wg_pad)

    out = pl.pallas_call(
        _ffn_body,
        grid=(E, NF, NS),
        in_specs=[
            pl.BlockSpec((SC, EPAD), lambda e, f, s: (s, 0)),
            pl.BlockSpec((S, D_MODEL), lambda e, f, s: (0, 0)),
            pl.BlockSpec((1, FC, D_MODEL), lambda e, f, s: (e, f, 0)),
            pl.BlockSpec((1, 1, FC), lambda e, f, s: (e * NF + f, 0, 0)),
            pl.BlockSpec((1, D_MODEL, FC), lambda e, f, s: (e, 0, f)),
            pl.BlockSpec((1, 1, D_MODEL), lambda e, f, s: (e, 0, 0)),
        ],
        out_specs=pl.BlockSpec((S, D_MODEL), lambda e, f, s: (0, 0)),
        out_shape=jax.ShapeDtypeStruct((S, D_MODEL), jnp.float32),
    )(wd, x2, w_up, bu3, w_down, bdn3)
    return out


def kernel(x, Wg, W_up, b_up, W_down, b_down):
    x2 = x.reshape(S, D_MODEL)
    wg_pad = jnp.zeros((EPAD, D_MODEL), jnp.float32).at[:E].set(Wg)
    bu3 = b_up.reshape(E * NF, 1, FC)
    bdn3 = b_down.reshape(E, 1, D_MODEL)
    out = _run(x2, wg_pad, W_up, bu3, W_down, bdn3)
    return out.reshape(1, S, D_MODEL)


# speedup vs baseline: 4.2807x; 2.4999x over previous
import functools

import jax
import jax.numpy as jnp
from jax import lax
from jax.experimental import pallas as pl
from jax.experimental.pallas import tpu as pltpu
from jax.experimental.pallas import tpu_sc as plsc

S = 2048
D = 768
DFF = 3072
E = 16
EPAD = 128
T = 128
NPAIR = 2 * S
MAXNT = NPAIR // T + E
DUMMYBLK = MAXNT
NROWS = (MAXNT + 1) * T
NREAL = MAXNT * T
TOKPAD = 7168

NSUB = 16
MCH = NPAIR // NSUB
NW = 32
GCH = NREAL // NW
CT = S // NW


def _router_body(x_ref, wg_ref, rt_ref):
    x = x_ref[...]
    wg = wg_ref[...]
    logits = lax.dot_general(
        x, wg, (((1,), (1,)), ((), ())), preferred_element_type=jnp.float32)
    col = lax.broadcasted_iota(jnp.int32, (S, EPAD), 1)
    neg = jnp.float32(-1e30)
    lm = jnp.where(col < E, logits, neg)
    m0 = jnp.max(lm, axis=1, keepdims=True)
    i0 = jnp.min(jnp.where(lm == m0, col, jnp.int32(EPAD)), axis=1,
                 keepdims=True)
    l2 = jnp.where(col == i0, neg, lm)
    m1 = jnp.max(l2, axis=1, keepdims=True)
    i1 = jnp.min(jnp.where(l2 == m1, col, jnp.int32(EPAD)), axis=1,
                 keepdims=True)
    z = jnp.exp(m1 - m0)
    denom = 1.0 + z
    w0 = 1.0 / denom
    w1 = z / denom
    rt_ref[...] = jnp.where(
        col == 0, i0.astype(jnp.float32),
        jnp.where(col == 1, i1.astype(jnp.float32),
                  jnp.where(col == 2, w0, jnp.where(col == 3, w1, 0.0))))


_meta_mesh = plsc.VectorSubcoreMesh(
    core_axis_name="c", subcore_axis_name="s", num_cores=1)


_SC_PARAMS = pltpu.CompilerParams(needs_layout_passes=False)


@functools.partial(
    pl.kernel,
    out_type=[
        jax.ShapeDtypeStruct((TOKPAD,), jnp.int32),
        jax.ShapeDtypeStruct((NROWS,), jnp.float32),
        jax.ShapeDtypeStruct((NPAIR,), jnp.int32),
        jax.ShapeDtypeStruct((MAXNT,), jnp.int32),
        jax.ShapeDtypeStruct((16,), jnp.int32),
    ],
    mesh=_meta_mesh,
    compiler_params=_SC_PARAMS,
    scratch_types=[
        pltpu.VMEM((MCH,), jnp.int32),
        pltpu.VMEM((MCH,), jnp.float32),
        pltpu.VMEM((MCH,), jnp.int32),
        pltpu.VMEM((MCH,), jnp.int32),
        pltpu.VMEM((2, 128), jnp.int32),
        pltpu.VMEM((16,), jnp.int32),
        pltpu.VMEM_SHARED((NSUB * 16,), jnp.int32),
        pltpu.VMEM((NSUB * 16,), jnp.int32),
        pltpu.VMEM((448,), jnp.int32),
        pltpu.VMEM((400,), jnp.float32),
        pltpu.VMEM((MAXNT,), jnp.int32),
        pltpu.VMEM((16,), jnp.int32),
        pltpu.SemaphoreType.DMA,
    ],
)
def _meta_kernel(e_hbm, w_hbm, tokin_hbm, tok_out, ws_out, pos_out, te_out,
                 nt_out, eid_v, w_v, tok_v, wr_v, pos2, cnt_v,
                 shcnt, allcnt_v, zbi, zbf, tev, ntv, sem):
    i32 = jnp.int32
    wid = lax.axis_index("s")
    lanes = lax.iota(i32, 16)
    zeros16 = jnp.zeros((16,), i32)
    zf16 = jnp.zeros((16,), jnp.float32)

    def bc(s):
        return jnp.broadcast_to(s, (16,))

    pltpu.sync_copy(e_hbm.at[pl.ds(wid * MCH, MCH)], eid_v)
    pltpu.sync_copy(w_hbm.at[pl.ds(wid * MCH, MCH)], w_v)
    pltpu.sync_copy(tokin_hbm.at[pl.ds(wid * MCH, MCH)], tok_v)

    def pass1(j, cnt):
        off = j * 16
        v = eid_v[pl.ds(off, 16)]
        rank = zeros16
        cadd = zeros16
        basel = zeros16
        for e in range(E):
            m = v == e
            mi = m.astype(i32)
            cs = plsc.cumsum(mi)
            rank = jnp.where(m, cs - 1, rank)
            ce = jnp.sum(jnp.where(lanes == e, cnt, zeros16))
            basel = basel + jnp.where(m, bc(ce), zeros16)
            cadd = cadd + jnp.where(lanes == e, bc(jnp.sum(mi)), zeros16)
        wr_v[pl.ds(off, 16)] = rank + basel
        return cnt + cadd

    cnt_v[...] = lax.fori_loop(0, MCH // 16, pass1, zeros16)

    pltpu.sync_copy(cnt_v, shcnt.at[pl.ds(wid * 16, 16)])
    plsc.subcore_barrier()
    pltpu.sync_copy(shcnt, allcnt_v)

    tot = zeros16
    prior = zeros16
    for r in range(NSUB):
        row = allcnt_v[pl.ds(r * 16, 16)]
        tot = tot + row
        prior = prior + jnp.where(bc(r < wid), row, zeros16)
    nt_e = (tot + (T - 1)) >> 7
    csum = plsc.cumsum(nt_e)
    base = (csum - nt_e) * T + prior
    nt_total = jnp.sum(nt_e)
    base_sc = [jnp.sum(jnp.where(lanes == e, base, zeros16))
               for e in range(E)]

    for j in range(MCH // 16):
        v = eid_v[pl.ds(j * 16, 16)]
        b = zeros16
        for e in range(E):
            b = b + jnp.where(v == e, bc(base_sc[e]), zeros16)
        pos2[j // 8, pl.ds((j % 8) * 16, 16)] = b + wr_v[pl.ds(j * 16, 16)]

    pltpu.sync_copy(pos2.at[0], pos_out.at[pl.ds(wid * MCH, 128)])
    pltpu.sync_copy(pos2.at[1], pos_out.at[pl.ds(wid * MCH + 128, 128)])

    for k in range(448 // 16):
        zbi[pl.ds(k * 16, 16)] = zeros16
    for k in range(400 // 16):
        zbf[pl.ds(k * 16, 16)] = zf16
    pltpu.sync_copy(zbi, tok_out.at[pl.ds(wid * 448, 448)])
    pltpu.sync_copy(zbf.at[pl.ds(0, NROWS // NSUB)],
                    ws_out.at[pl.ds(wid * (NROWS // NSUB), NROWS // NSUB)])
    plsc.subcore_barrier()

    for r in range(2):
        pltpu.async_copy(
            tok_v.at[pl.ds(r * 128, 128)], tok_out.at[pos2.at[r]], sem).wait()
        pltpu.async_copy(
            w_v.at[pl.ds(r * 128, 128)], ws_out.at[pos2.at[r]], sem).wait()

    @pl.when(wid == 0)
    def _():
        csum_sc = [jnp.sum(jnp.where(lanes == e, csum, zeros16))
                   for e in range(E)]
        ntb = bc(nt_total)
        for g in range(MAXNT // 16):
            tv = lanes + g * 16
            expt = zeros16
            for e in range(E):
                expt = expt + (tv >= bc(csum_sc[e])).astype(i32)
            tev[pl.ds(g * 16, 16)] = jnp.where(
                tv < ntb, expt, jnp.full((16,), E - 1, i32))
        pltpu.sync_copy(tev, te_out)
        ntv[...] = ntb
        pltpu.sync_copy(ntv, nt_out)


def _gmm_body(te_ref, nt_ref, tok_ref, x_any, wu_ref, bu_ref, wdn_ref,
              bdn_ref, ws_ref, y_ref, xs_buf, sem0, sem1):
    t = pl.program_id(0)
    nt = nt_ref[0]
    sems = [sem0, sem1]

    def issue(tile, slot, sem):
        base = tile * T
        for r in range(T):
            tok = tok_ref[base + r]
            pltpu.make_async_copy(
                x_any.at[pl.ds(tok, 1), :],
                xs_buf.at[slot, pl.ds(r, 1), :],
                sem).start()

    @pl.when(t == 0)
    def _():
        issue(0, 0, sems[0])

    @pl.when((t + 1) < nt)
    def _():
        for s in range(2):
            @pl.when((t + 1) % 2 == s)
            def _():
                issue(t + 1, s, sems[s])

    @pl.when(t < nt)
    def _():
        for s in range(2):
            @pl.when(t % 2 == s)
            def _():
                pltpu.make_async_copy(
                    x_any.at[pl.ds(0, T), :], xs_buf.at[s], sems[s]).wait()
                xs = xs_buf[s].astype(jnp.bfloat16)
                wu = wu_ref[0].astype(jnp.bfloat16)
                h_pre = lax.dot_general(
                    xs, wu, (((1,), (1,)), ((), ())),
                    preferred_element_type=jnp.float32)
                h_pre = h_pre + bu_ref[0, 0, :][None, :]
                h = 0.5 * h_pre * (
                    1.0 + lax.erf(h_pre * 0.7071067811865476))
                wdn = wdn_ref[0].astype(jnp.bfloat16)
                y = lax.dot_general(
                    h.astype(jnp.bfloat16), wdn, (((1,), (1,)), ((), ())),
                    preferred_element_type=jnp.float32)
                y = y + bdn_ref[0, 0, :][None, :]
                y_ref[...] = ws_ref[0] * y


_full_mesh = plsc.VectorSubcoreMesh(core_axis_name="c", subcore_axis_name="s")


@functools.partial(
    pl.kernel,
    out_type=jax.ShapeDtypeStruct((S, D), jnp.float32),
    mesh=_full_mesh,
    compiler_params=_SC_PARAMS,
    scratch_types=[
        pltpu.VMEM((CT,), jnp.int32),
        pltpu.VMEM((CT,), jnp.int32),
        pltpu.VMEM((CT, D), jnp.float32),
        pltpu.VMEM((CT, D), jnp.float32),
        pltpu.SemaphoreType.DMA,
    ],
)
def _combine_kernel(pos_hbm, y_hbm, out_hbm, p0, p1, a0, a1, sem):
    wid = lax.axis_index("s") * 2 + lax.axis_index("c")
    pltpu.sync_copy(pos_hbm.at[pl.ds(wid * CT, CT)], p0)
    pltpu.sync_copy(pos_hbm.at[pl.ds(S + wid * CT, CT)], p1)
    pltpu.async_copy(y_hbm.at[p0], a0, sem).wait()
    pltpu.async_copy(y_hbm.at[p1], a1, sem).wait()

    def row(r, carry):
        for c in range(D // 16):
            sl = pl.ds(c * 16, 16)
            a0[r, sl] = a0[r, sl] + a1[r, sl]
        return carry

    lax.fori_loop(0, CT, row, 0)
    pltpu.sync_copy(a0, out_hbm.at[pl.ds(wid * CT, CT)])


@jax.jit
def _run(x2, wg_pad, w_up, bu3, w_down, bdn3):
    rt = pl.pallas_call(
        _router_body,
        out_shape=jax.ShapeDtypeStruct((S, EPAD), jnp.float32),
    )(x2, wg_pad)

    e_all = jnp.concatenate(
        [rt[:, 0].astype(jnp.int32), rt[:, 1].astype(jnp.int32)])
    w_all = jnp.concatenate([rt[:, 2], rt[:, 3]])
    tok_ramp = jnp.arange(NPAIR, dtype=jnp.int32) & (S - 1)

    tok_sorted, w_sorted, pos_all, tile_expert, nt16 = _meta_kernel(
        e_all, w_all, tok_ramp)

    ws3 = w_sorted.reshape(NROWS // T, T, 1)

    y = pl.pallas_call(
        _gmm_body,
        grid_spec=pltpu.PrefetchScalarGridSpec(
            num_scalar_prefetch=3,
            grid=(MAXNT,),
            in_specs=[
                pl.BlockSpec(memory_space=pltpu.MemorySpace.HBM),
                pl.BlockSpec((1, DFF, D), lambda t, te, nt, tk: (te[t], 0, 0)),
                pl.BlockSpec((1, 1, DFF), lambda t, te, nt, tk: (te[t], 0, 0)),
                pl.BlockSpec((1, D, DFF), lambda t, te, nt, tk: (te[t], 0, 0)),
                pl.BlockSpec((1, 1, D), lambda t, te, nt, tk: (te[t], 0, 0)),
                pl.BlockSpec(
                    (1, T, 1),
                    lambda t, te, nt, tk: (jnp.where(t < nt[0], t, DUMMYBLK),
                                           0, 0)),
            ],
            out_specs=pl.BlockSpec(
                (T, D),
                lambda t, te, nt, tk: (jnp.where(t < nt[0], t, DUMMYBLK), 0)),
            scratch_shapes=[
                pltpu.VMEM((2, T, D), jnp.float32),
                pltpu.SemaphoreType.DMA,
                pltpu.SemaphoreType.DMA,
            ],
        ),
        out_shape=jax.ShapeDtypeStruct((NROWS, D), jnp.float32),
    )(tile_expert, nt16, tok_sorted, x2, w_up, bu3, w_down, bdn3, ws3)

    return _combine_kernel(pos_all, y)


def kernel(x, Wg, W_up, b_up, W_down, b_down):
    x2 = x.reshape(S, D)
    wg_pad = jnp.zeros((EPAD, D), jnp.float32).at[:E].set(Wg)
    bu3 = b_up.reshape(E, 1, DFF)
    bdn3 = b_down.reshape(E, 1, D)
    out = _run(x2, wg_pad, W_up, bu3, W_down, bdn3)
    return out.reshape(1, S, D)
